# Initial kernel scaffold; baseline (speedup 1.0000x reference)
import jax
import jax.numpy as jnp
from jax.experimental import pallas as pl


def kernel(users, items, user_emb_weight, item_emb_weight, edge_index, graph_values):
    raise NotImplementedError("write your pallas kernel here")


# speedup vs baseline: 4.4298x; 4.4298x over previous
import jax
import jax.numpy as jnp
from jax import lax
from jax.experimental import pallas as pl
from jax.experimental.pallas import tpu as pltpu
from jax.experimental.pallas import tpu_sc as plsc

N_USERS = 25000
N_ITEMS = 25000
N_NODES = N_USERS + N_ITEMS
N_EDGES = 800000
HALF = 32
BATCH = 16384

NC = 2
NS = 16
CHUNK = 128
SLAB = 4
E_SLAB = CHUNK * SLAB
N_EDGES_PAD = 819200
N_CHUNK_ROWS = N_EDGES_PAD // CHUNK
N_SLABS = N_EDGES_PAD // E_SLAB
SLABS_SUB = N_SLABS // NS
PIECE = 400
N_PIECES = N_NODES // PIECE
P_SUB = BATCH // NS
PCHUNK = 64
P_LOOPS = P_SUB // PCHUNK


def _sc_body(init_ref, rows_ref, cols_ref, vals_ref, users_ref, items_ref,
             gamma_ref, l1_ref, l2_ref, l3_ref,
             acc, colsv, rowsv, valsv, gath, uv, iv, gammav, gsem, psem):
  c = lax.axis_index("c")
  s = lax.axis_index("s")

  def propagate(src_tbl, dst_tbl):
    @pl.loop(0, PIECE)
    def _(r):
      gath[r, pl.ds(0, 16)] = jnp.zeros((16,), jnp.float32)
      gath[r, pl.ds(16, 16)] = jnp.zeros((16,), jnp.float32)

    @pl.loop(s, N_PIECES, step=NS)
    def _(j):
      pltpu.sync_copy(gath.at[pl.ds(0, PIECE)], acc.at[pl.ds(j * PIECE, PIECE)])
    plsc.subcore_barrier()

    @pl.loop(0, SLABS_SUB)
    def _(jj):
      j = jj * NS + s
      pltpu.sync_copy(cols_ref.at[pl.ds(j * SLAB, SLAB)], colsv)
      pltpu.sync_copy(rows_ref.at[pl.ds(j * SLAB, SLAB)], rowsv)
      pltpu.sync_copy(vals_ref.at[pl.ds(j * SLAB, SLAB)], valsv)
      descs = [
          pltpu.async_copy(src_tbl.at[colsv.at[k]],
                           gath.at[pl.ds(k * CHUNK, CHUNK)], gsem.at[k])
          for k in range(SLAB)
      ]
      for k in range(SLAB):
        descs[k].wait()
        kb = k * CHUNK

        @pl.loop(0, CHUNK)
        def _(e, kb=kb, k=k):
          kidx = jnp.full((16,), k, jnp.int32)
          eidx = jnp.full((16,), e, jnp.int32)
          v = plsc.load_gather(valsv, [kidx, eidx])
          gath[kb + e, pl.ds(0, 16)] = gath[kb + e, pl.ds(0, 16)] * v
          gath[kb + e, pl.ds(16, 16)] = gath[kb + e, pl.ds(16, 16)] * v

        pltpu.sync_copy(gath.at[pl.ds(kb, CHUNK)], acc.at[rowsv.at[k]],
                        add=True)

    plsc.subcore_barrier()

    @pl.loop(s, N_PIECES, step=NS)
    def _(j):
      pltpu.sync_copy(acc.at[pl.ds(j * PIECE, PIECE)],
                      dst_tbl.at[pl.ds(j * PIECE, PIECE)])

  t0 = init_ref.at[c]
  t1 = l1_ref.at[c]
  t2 = l2_ref.at[c]
  t3 = l3_ref.at[c]
  propagate(t0, t1)
  plsc.subcore_barrier()
  propagate(t1, t2)
  plsc.subcore_barrier()
  propagate(t2, t3)
  plsc.subcore_barrier()

  tables = (t0, t1, t2, t3)
  for p in range(P_LOOPS):
    base = s * P_SUB + p * PCHUNK
    pltpu.sync_copy(users_ref.at[pl.ds(base, PCHUNK)], uv)
    pltpu.sync_copy(items_ref.at[pl.ds(base, PCHUNK)], iv)

    @pl.loop(0, PCHUNK, step=16)
    def _(t):
      iv[pl.ds(t, 16)] = iv[pl.ds(t, 16)] + N_USERS

    descs = []
    for t in range(4):
      descs.append(pltpu.async_copy(
          tables[t].at[uv], gath.at[pl.ds(t * PCHUNK, PCHUNK)],
          psem.at[2 * t]))
      descs.append(pltpu.async_copy(
          tables[t].at[iv], gath.at[pl.ds(4 * PCHUNK + t * PCHUNK, PCHUNK)],
          psem.at[2 * t + 1]))
    for d_ in descs:
      d_.wait()

    @pl.loop(0, PCHUNK)
    def _(e, p=p):
      ulo = (gath[0 * PCHUNK + e, pl.ds(0, 16)] +
             gath[1 * PCHUNK + e, pl.ds(0, 16)] +
             gath[2 * PCHUNK + e, pl.ds(0, 16)] +
             gath[3 * PCHUNK + e, pl.ds(0, 16)])
      uhi = (gath[0 * PCHUNK + e, pl.ds(16, 16)] +
             gath[1 * PCHUNK + e, pl.ds(16, 16)] +
             gath[2 * PCHUNK + e, pl.ds(16, 16)] +
             gath[3 * PCHUNK + e, pl.ds(16, 16)])
      ilo = (gath[4 * PCHUNK + e, pl.ds(0, 16)] +
             gath[5 * PCHUNK + e, pl.ds(0, 16)] +
             gath[6 * PCHUNK + e, pl.ds(0, 16)] +
             gath[7 * PCHUNK + e, pl.ds(0, 16)])
      ihi = (gath[4 * PCHUNK + e, pl.ds(16, 16)] +
             gath[5 * PCHUNK + e, pl.ds(16, 16)] +
             gath[6 * PCHUNK + e, pl.ds(16, 16)] +
             gath[7 * PCHUNK + e, pl.ds(16, 16)])
      prod = ulo * ilo + uhi * ihi
      cs = plsc.cumsum(prod)
      lane = lax.broadcasted_iota(jnp.int32, (16,), 0)
      plsc.store_scatter(gammav,
                         [jnp.full((16,), p * PCHUNK + e, jnp.int32)],
                         cs, mask=lane == 15)

  pltpu.sync_copy(gammav, gamma_ref.at[c, pl.ds(s * P_SUB, P_SUB)])


_SCRATCH = [
    pltpu.VMEM_SHARED((N_NODES, HALF), jnp.float32),
    pltpu.VMEM((SLAB, CHUNK), jnp.int32),
    pltpu.VMEM((SLAB, CHUNK), jnp.int32),
    pltpu.VMEM((SLAB, CHUNK), jnp.float32),
    pltpu.VMEM((E_SLAB, HALF), jnp.float32),
    pltpu.VMEM((PCHUNK,), jnp.int32),
    pltpu.VMEM((PCHUNK,), jnp.int32),
    pltpu.VMEM((P_SUB,), jnp.float32),
    pltpu.SemaphoreType.DMA((SLAB,)),
    pltpu.SemaphoreType.DMA((8,)),
]

_OUT = (
    jax.ShapeDtypeStruct((NC, BATCH), jnp.float32),
    jax.ShapeDtypeStruct((NC, N_NODES, HALF), jnp.float32),
    jax.ShapeDtypeStruct((NC, N_NODES, HALF), jnp.float32),
    jax.ShapeDtypeStruct((NC, N_NODES, HALF), jnp.float32),
)


def _combine_body(p_ref, o_ref):
  o_ref[...] = (p_ref[0] + p_ref[1]) * jnp.float32(1.0 / 16.0)


def kernel(users, items, user_emb_weight, item_emb_weight, edge_index,
           graph_values):
  all_emb = jnp.concatenate([user_emb_weight, item_emb_weight], axis=0)
  init = jnp.stack([all_emb[:, :HALF], all_emb[:, HALF:]])
  pad = N_EDGES_PAD - N_EDGES
  rows = jnp.concatenate(
      [edge_index[0], jnp.zeros((pad,), jnp.int32)]).reshape(
          N_CHUNK_ROWS, CHUNK)
  cols = jnp.concatenate(
      [edge_index[1], jnp.zeros((pad,), jnp.int32)]).reshape(
          N_CHUNK_ROWS, CHUNK)
  vals = jnp.concatenate(
      [graph_values, jnp.zeros((pad,), jnp.float32)]).reshape(
          N_CHUNK_ROWS, CHUNK)

  mesh = plsc.VectorSubcoreMesh(core_axis_name="c", subcore_axis_name="s",
                                num_cores=NC, num_subcores=NS)
  sc = pl.kernel(_sc_body, out_type=_OUT, mesh=mesh, scratch_types=_SCRATCH,
                 compiler_params=pltpu.CompilerParams(
                     needs_layout_passes=False,
                     use_tc_tiling_on_sc=False))
  gamma_p, _, _, _ = sc(init, rows, cols, vals, users, items)

  out = pl.pallas_call(
      _combine_body,
      out_shape=jax.ShapeDtypeStruct((128, 128), jnp.float32))(
          gamma_p.reshape(NC, 128, 128))
  return out.reshape(BATCH)


# speedup vs baseline: 8.5423x; 1.9284x over previous
import jax
import jax.numpy as jnp
from jax import lax
from jax.experimental import pallas as pl
from jax.experimental.pallas import tpu as pltpu
from jax.experimental.pallas import tpu_sc as plsc

N_USERS = 25000
N_ITEMS = 25000
N_NODES = N_USERS + N_ITEMS
N_EDGES = 800000
HALF = 32
BATCH = 16384

NC = 2
NS = 16
CHUNK = 128
SLAB = 5
E_SLAB = CHUNK * SLAB
N_EDGES_PAD = 819200
N_SLABS = N_EDGES_PAD // E_SLAB
SLABS_SUB = N_SLABS // NS
PIECE = 400
N_PIECES = N_NODES // PIECE
P_SUB = BATCH // NS
PCHUNK = 64
P_LOOPS = P_SUB // PCHUNK


def _sc_body(init_ref, packed_ref, users_ref, items_ref,
             gamma_ref, l1_ref, l2_ref, l3_ref,
             acc, pbufa, pbufb, gath, uv, iv, gammav, gsem, ssem, isem):
  c = lax.axis_index("c")
  s = lax.axis_index("s")

  def propagate(src_tbl, dst_tbl):
    @pl.loop(0, PIECE)
    def _(r):
      gath[r, pl.ds(0, 16)] = jnp.zeros((16,), jnp.float32)
      gath[r, pl.ds(16, 16)] = jnp.zeros((16,), jnp.float32)

    @pl.loop(s, N_PIECES, step=NS)
    def _(j):
      pltpu.sync_copy(gath.at[pl.ds(0, PIECE)],
                      acc.at[pl.ds(j * PIECE, PIECE)])
    plsc.subcore_barrier()

    pltpu.sync_copy(packed_ref.at[s], pbufa)

    def do_slab(cur, nxt, jj):
      j = jj * NS + s

      for k in range(SLAB):
        @pl.when(jj > 0)
        def _(k=k):
          pltpu.make_async_copy(
              gath.at[pl.ds(k * CHUNK, CHUNK)],
              acc.at[cur.at[1, k]], ssem.at[k]).wait()
        pltpu.async_copy(src_tbl.at[cur.at[0, k]],
                         gath.at[pl.ds(k * CHUNK, CHUNK)], gsem.at[k])

      @pl.when(jj + 1 < SLABS_SUB)
      def _():
        pltpu.async_copy(packed_ref.at[(jj + 1) * NS + s], nxt, isem)

      for k in range(SLAB):
        pltpu.make_async_copy(src_tbl.at[cur.at[0, k]],
                              gath.at[pl.ds(k * CHUNK, CHUNK)],
                              gsem.at[k]).wait()
        kb = k * CHUNK
        k2 = jnp.full((16,), 2, jnp.int32)
        kk = jnp.full((16,), k, jnp.int32)

        @plsc.parallel_loop(0, CHUNK, unroll=4)
        def _(e, kb=kb, k2=k2, kk=kk):
          v = plsc.bitcast(
              plsc.load_gather(cur, [k2, kk, jnp.full((16,), e, jnp.int32)]),
              jnp.float32)
          gath[kb + e, pl.ds(0, 16)] = gath[kb + e, pl.ds(0, 16)] * v
          gath[kb + e, pl.ds(16, 16)] = gath[kb + e, pl.ds(16, 16)] * v

        pltpu.async_copy(gath.at[pl.ds(kb, CHUNK)], acc.at[cur.at[1, k]],
                         ssem.at[k], add=True)

      @pl.when(jj + 1 < SLABS_SUB)
      def _():
        pltpu.make_async_copy(packed_ref.at[(jj + 1) * NS + s], nxt,
                              isem).wait()

    @pl.loop(0, SLABS_SUB // 2)
    def _(m):
      do_slab(pbufa, pbufb, 2 * m)
      do_slab(pbufb, pbufa, 2 * m + 1)

    for k in range(SLAB):
      pltpu.make_async_copy(gath.at[pl.ds(k * CHUNK, CHUNK)],
                            acc.at[pbufb.at[1, k]], ssem.at[k]).wait()

    plsc.subcore_barrier()

    @pl.loop(s, N_PIECES, step=NS)
    def _(j):
      pltpu.sync_copy(acc.at[pl.ds(j * PIECE, PIECE)],
                      dst_tbl.at[pl.ds(j * PIECE, PIECE)])

  t0 = init_ref.at[c]
  t1 = l1_ref.at[c]
  t2 = l2_ref.at[c]
  t3 = l3_ref.at[c]
  propagate(t0, t1)
  plsc.subcore_barrier()
  propagate(t1, t2)
  plsc.subcore_barrier()
  propagate(t2, t3)
  plsc.subcore_barrier()

  tables = (t0, t1, t2, t3)
  for p in range(P_LOOPS):
    base = s * P_SUB + p * PCHUNK
    pltpu.sync_copy(users_ref.at[pl.ds(base, PCHUNK)], uv)
    pltpu.sync_copy(items_ref.at[pl.ds(base, PCHUNK)], iv)

    @pl.loop(0, PCHUNK, step=16)
    def _(t):
      iv[pl.ds(t, 16)] = iv[pl.ds(t, 16)] + N_USERS

    descs = []
    for t in range(4):
      descs.append(pltpu.async_copy(
          tables[t].at[uv], gath.at[pl.ds(t * PCHUNK, PCHUNK)],
          gsem.at[t % SLAB]))
      descs.append(pltpu.async_copy(
          tables[t].at[iv], gath.at[pl.ds(4 * PCHUNK + t * PCHUNK, PCHUNK)],
          ssem.at[t % SLAB]))
    for d_ in descs:
      d_.wait()

    @pl.loop(0, PCHUNK)
    def _(e, p=p):
      ulo = (gath[0 * PCHUNK + e, pl.ds(0, 16)] +
             gath[1 * PCHUNK + e, pl.ds(0, 16)] +
             gath[2 * PCHUNK + e, pl.ds(0, 16)] +
             gath[3 * PCHUNK + e, pl.ds(0, 16)])
      uhi = (gath[0 * PCHUNK + e, pl.ds(16, 16)] +
             gath[1 * PCHUNK + e, pl.ds(16, 16)] +
             gath[2 * PCHUNK + e, pl.ds(16, 16)] +
             gath[3 * PCHUNK + e, pl.ds(16, 16)])
      ilo = (gath[4 * PCHUNK + e, pl.ds(0, 16)] +
             gath[5 * PCHUNK + e, pl.ds(0, 16)] +
             gath[6 * PCHUNK + e, pl.ds(0, 16)] +
             gath[7 * PCHUNK + e, pl.ds(0, 16)])
      ihi = (gath[4 * PCHUNK + e, pl.ds(16, 16)] +
             gath[5 * PCHUNK + e, pl.ds(16, 16)] +
             gath[6 * PCHUNK + e, pl.ds(16, 16)] +
             gath[7 * PCHUNK + e, pl.ds(16, 16)])
      prod = ulo * ilo + uhi * ihi
      cs = plsc.cumsum(prod)
      lane = lax.broadcasted_iota(jnp.int32, (16,), 0)
      plsc.store_scatter(gammav,
                         [jnp.full((16,), p * PCHUNK + e, jnp.int32)],
                         cs, mask=lane == 15)

  pltpu.sync_copy(gammav, gamma_ref.at[c, pl.ds(s * P_SUB, P_SUB)])


_SCRATCH = [
    pltpu.VMEM_SHARED((N_NODES, HALF), jnp.float32),
    pltpu.VMEM((3, SLAB, CHUNK), jnp.int32),
    pltpu.VMEM((3, SLAB, CHUNK), jnp.int32),
    pltpu.VMEM((E_SLAB, HALF), jnp.float32),
    pltpu.VMEM((PCHUNK,), jnp.int32),
    pltpu.VMEM((PCHUNK,), jnp.int32),
    pltpu.VMEM((P_SUB,), jnp.float32),
    pltpu.SemaphoreType.DMA((SLAB,)),
    pltpu.SemaphoreType.DMA((SLAB,)),
    pltpu.SemaphoreType.DMA,
]

_OUT = (
    jax.ShapeDtypeStruct((NC, BATCH), jnp.float32),
    jax.ShapeDtypeStruct((NC, N_NODES, HALF), jnp.float32),
    jax.ShapeDtypeStruct((NC, N_NODES, HALF), jnp.float32),
    jax.ShapeDtypeStruct((NC, N_NODES, HALF), jnp.float32),
)


def _combine_body(p_ref, o_ref):
  o_ref[...] = (p_ref[0] + p_ref[1]) * jnp.float32(1.0 / 16.0)


def kernel(users, items, user_emb_weight, item_emb_weight, edge_index,
           graph_values):
  all_emb = jnp.concatenate([user_emb_weight, item_emb_weight], axis=0)
  init = jnp.stack([all_emb[:, :HALF], all_emb[:, HALF:]])
  pad = N_EDGES_PAD - N_EDGES
  cols = jnp.concatenate(
      [edge_index[1], jnp.zeros((pad,), jnp.int32)]).reshape(
          N_SLABS, SLAB, CHUNK)
  rows = jnp.concatenate(
      [edge_index[0], jnp.zeros((pad,), jnp.int32)]).reshape(
          N_SLABS, SLAB, CHUNK)
  vals = lax.bitcast_convert_type(
      jnp.concatenate([graph_values, jnp.zeros((pad,), jnp.float32)]),
      jnp.int32).reshape(N_SLABS, SLAB, CHUNK)
  packed = jnp.stack([cols, rows, vals], axis=1)

  mesh = plsc.VectorSubcoreMesh(core_axis_name="c", subcore_axis_name="s",
                                num_cores=NC, num_subcores=NS)
  sc = pl.kernel(_sc_body, out_type=_OUT, mesh=mesh, scratch_types=_SCRATCH,
                 compiler_params=pltpu.CompilerParams(
                     needs_layout_passes=False,
                     use_tc_tiling_on_sc=False))
  gamma_p, _, _, _ = sc(init, packed, users, items)

  out = pl.pallas_call(
      _combine_body,
      out_shape=jax.ShapeDtypeStruct((128, 128), jnp.float32))(
          gamma_p.reshape(NC, 128, 128))
  return out.reshape(BATCH)


# speedup vs baseline: 8.5522x; 1.0012x over previous
import jax
import jax.numpy as jnp
from jax import lax
from jax.experimental import pallas as pl
from jax.experimental.pallas import tpu as pltpu
from jax.experimental.pallas import tpu_sc as plsc

N_USERS = 25000
N_ITEMS = 25000
N_NODES = N_USERS + N_ITEMS
N_EDGES = 800000
HALF = 32
BATCH = 16384

NC = 2
NS = 16
CHUNK = 128
SLAB = 5
E_SLAB = CHUNK * SLAB
N_EDGES_PAD = 819200
N_SLABS = N_EDGES_PAD // E_SLAB
SLABS_SUB = N_SLABS // NS
PIECE = 400
N_PIECES = N_NODES // PIECE
P_SUB = BATCH // NS
PCHUNK = 64
P_LOOPS = P_SUB // PCHUNK


def _sc_body(init_ref, packed_ref, users_ref, items_ref,
             gamma_ref, l1_ref, l2_ref, l3_ref,
             acc, pbufa, pbufb, gath, uv, iv, gammav, gsem, ssem, isem):
  c = lax.axis_index("c")
  s = lax.axis_index("s")

  def propagate(src_tbl, dst_tbl):
    @pl.loop(0, PIECE)
    def _(r):
      gath[r, pl.ds(0, 16)] = jnp.zeros((16,), jnp.float32)
      gath[r, pl.ds(16, 16)] = jnp.zeros((16,), jnp.float32)

    @pl.loop(s, N_PIECES, step=NS)
    def _(j):
      pltpu.sync_copy(gath.at[pl.ds(0, PIECE)],
                      acc.at[pl.ds(j * PIECE, PIECE)])
    plsc.subcore_barrier()

    pltpu.sync_copy(packed_ref.at[s], pbufa)

    def do_slab(cur, nxt, jj):
      j = jj * NS + s

      for k in range(SLAB):
        @pl.when(jj > 0)
        def _(k=k):
          pltpu.make_async_copy(
              gath.at[pl.ds(k * CHUNK, CHUNK)],
              acc.at[cur.at[1, k]], ssem.at[k]).wait()
        pltpu.async_copy(src_tbl.at[cur.at[0, k]],
                         gath.at[pl.ds(k * CHUNK, CHUNK)], gsem.at[k])

      @pl.when(jj + 1 < SLABS_SUB)
      def _():
        pltpu.async_copy(packed_ref.at[(jj + 1) * NS + s], nxt, isem)

      for k in range(SLAB):
        pltpu.make_async_copy(src_tbl.at[cur.at[0, k]],
                              gath.at[pl.ds(k * CHUNK, CHUNK)],
                              gsem.at[k]).wait()
        kb = k * CHUNK
        k2 = jnp.full((16,), 2, jnp.int32)
        kk = jnp.full((16,), k, jnp.int32)

        @plsc.parallel_loop(0, CHUNK, unroll=8)
        def _(e, kb=kb, k2=k2, kk=kk):
          v = plsc.bitcast(
              plsc.load_gather(cur, [k2, kk, jnp.full((16,), e, jnp.int32)]),
              jnp.float32)
          gath[kb + e, pl.ds(0, 16)] = gath[kb + e, pl.ds(0, 16)] * v
          gath[kb + e, pl.ds(16, 16)] = gath[kb + e, pl.ds(16, 16)] * v

        pltpu.async_copy(gath.at[pl.ds(kb, CHUNK)], acc.at[cur.at[1, k]],
                         ssem.at[k], add=True)

      @pl.when(jj + 1 < SLABS_SUB)
      def _():
        pltpu.make_async_copy(packed_ref.at[(jj + 1) * NS + s], nxt,
                              isem).wait()

    @pl.loop(0, SLABS_SUB // 2)
    def _(m):
      do_slab(pbufa, pbufb, 2 * m)
      do_slab(pbufb, pbufa, 2 * m + 1)

    for k in range(SLAB):
      pltpu.make_async_copy(gath.at[pl.ds(k * CHUNK, CHUNK)],
                            acc.at[pbufb.at[1, k]], ssem.at[k]).wait()

    plsc.subcore_barrier()

    @pl.loop(s, N_PIECES, step=NS)
    def _(j):
      pltpu.sync_copy(acc.at[pl.ds(j * PIECE, PIECE)],
                      dst_tbl.at[pl.ds(j * PIECE, PIECE)])

  t0 = init_ref.at[c]
  t1 = l1_ref.at[c]
  t2 = l2_ref.at[c]
  t3 = l3_ref.at[c]
  propagate(t0, t1)
  plsc.subcore_barrier()
  propagate(t1, t2)
  plsc.subcore_barrier()
  propagate(t2, t3)
  plsc.subcore_barrier()

  tables = (t0, t1, t2, t3)
  for p in range(P_LOOPS):
    base = s * P_SUB + p * PCHUNK
    pltpu.sync_copy(users_ref.at[pl.ds(base, PCHUNK)], uv)
    pltpu.sync_copy(items_ref.at[pl.ds(base, PCHUNK)], iv)

    @pl.loop(0, PCHUNK, step=16)
    def _(t):
      iv[pl.ds(t, 16)] = iv[pl.ds(t, 16)] + N_USERS

    descs = []
    for t in range(4):
      descs.append(pltpu.async_copy(
          tables[t].at[uv], gath.at[pl.ds(t * PCHUNK, PCHUNK)],
          gsem.at[t % SLAB]))
      descs.append(pltpu.async_copy(
          tables[t].at[iv], gath.at[pl.ds(4 * PCHUNK + t * PCHUNK, PCHUNK)],
          ssem.at[t % SLAB]))
    for d_ in descs:
      d_.wait()

    @pl.loop(0, PCHUNK)
    def _(e, p=p):
      ulo = (gath[0 * PCHUNK + e, pl.ds(0, 16)] +
             gath[1 * PCHUNK + e, pl.ds(0, 16)] +
             gath[2 * PCHUNK + e, pl.ds(0, 16)] +
             gath[3 * PCHUNK + e, pl.ds(0, 16)])
      uhi = (gath[0 * PCHUNK + e, pl.ds(16, 16)] +
             gath[1 * PCHUNK + e, pl.ds(16, 16)] +
             gath[2 * PCHUNK + e, pl.ds(16, 16)] +
             gath[3 * PCHUNK + e, pl.ds(16, 16)])
      ilo = (gath[4 * PCHUNK + e, pl.ds(0, 16)] +
             gath[5 * PCHUNK + e, pl.ds(0, 16)] +
             gath[6 * PCHUNK + e, pl.ds(0, 16)] +
             gath[7 * PCHUNK + e, pl.ds(0, 16)])
      ihi = (gath[4 * PCHUNK + e, pl.ds(16, 16)] +
             gath[5 * PCHUNK + e, pl.ds(16, 16)] +
             gath[6 * PCHUNK + e, pl.ds(16, 16)] +
             gath[7 * PCHUNK + e, pl.ds(16, 16)])
      prod = ulo * ilo + uhi * ihi
      cs = plsc.cumsum(prod)
      lane = lax.broadcasted_iota(jnp.int32, (16,), 0)
      plsc.store_scatter(gammav,
                         [jnp.full((16,), p * PCHUNK + e, jnp.int32)],
                         cs, mask=lane == 15)

  pltpu.sync_copy(gammav, gamma_ref.at[c, pl.ds(s * P_SUB, P_SUB)])


_SCRATCH = [
    pltpu.VMEM_SHARED((N_NODES, HALF), jnp.float32),
    pltpu.VMEM((3, SLAB, CHUNK), jnp.int32),
    pltpu.VMEM((3, SLAB, CHUNK), jnp.int32),
    pltpu.VMEM((E_SLAB, HALF), jnp.float32),
    pltpu.VMEM((PCHUNK,), jnp.int32),
    pltpu.VMEM((PCHUNK,), jnp.int32),
    pltpu.VMEM((P_SUB,), jnp.float32),
    pltpu.SemaphoreType.DMA((SLAB,)),
    pltpu.SemaphoreType.DMA((SLAB,)),
    pltpu.SemaphoreType.DMA,
]

_OUT = (
    jax.ShapeDtypeStruct((NC, BATCH), jnp.float32),
    jax.ShapeDtypeStruct((NC, N_NODES, HALF), jnp.float32),
    jax.ShapeDtypeStruct((NC, N_NODES, HALF), jnp.float32),
    jax.ShapeDtypeStruct((NC, N_NODES, HALF), jnp.float32),
)


def _combine_body(p_ref, o_ref):
  o_ref[...] = (p_ref[0] + p_ref[1]) * jnp.float32(1.0 / 16.0)


def kernel(users, items, user_emb_weight, item_emb_weight, edge_index,
           graph_values):
  all_emb = jnp.concatenate([user_emb_weight, item_emb_weight], axis=0)
  init = jnp.stack([all_emb[:, :HALF], all_emb[:, HALF:]])
  pad = N_EDGES_PAD - N_EDGES
  cols = jnp.concatenate(
      [edge_index[1], jnp.zeros((pad,), jnp.int32)]).reshape(
          N_SLABS, SLAB, CHUNK)
  rows = jnp.concatenate(
      [edge_index[0], jnp.zeros((pad,), jnp.int32)]).reshape(
          N_SLABS, SLAB, CHUNK)
  vals = lax.bitcast_convert_type(
      jnp.concatenate([graph_values, jnp.zeros((pad,), jnp.float32)]),
      jnp.int32).reshape(N_SLABS, SLAB, CHUNK)
  packed = jnp.stack([cols, rows, vals], axis=1)

  mesh = plsc.VectorSubcoreMesh(core_axis_name="c", subcore_axis_name="s",
                                num_cores=NC, num_subcores=NS)
  sc = pl.kernel(_sc_body, out_type=_OUT, mesh=mesh, scratch_types=_SCRATCH,
                 compiler_params=pltpu.CompilerParams(
                     needs_layout_passes=False,
                     use_tc_tiling_on_sc=False))
  gamma_p, _, _, _ = sc(init, packed, users, items)

  out = pl.pallas_call(
      _combine_body,
      out_shape=jax.ShapeDtypeStruct((128, 128), jnp.float32))(
          gamma_p.reshape(NC, 128, 128))
  return out.reshape(BATCH)
